# Initial kernel scaffold; baseline (speedup 1.0000x reference)
import jax
import jax.numpy as jnp
from jax import lax
from jax.experimental import pallas as pl
from jax.experimental.pallas import tpu as pltpu

_B, _N, _D = 1024, 200, 128
_HIDDEN = 128
_INPUT_DIM = 256
_MAX_F = 200
_N_SESSIONS = 1000
_G = 8
_SCALE = float(_HIDDEN) ** 0.5


def _body(eid_ref, inp_ref, ts_ref, tokW_ref, tokb_ref, projW_ref, projb_ref,
          posmod_ref, sess_ref, x_ref, emb_ref):
    i = pl.program_id(0)
    inp = inp_ref[...].reshape(_G * _N, _D)
    t = jnp.dot(inp, tokW_ref[...], preferred_element_type=jnp.float32)
    t = t + tokb_ref[...]
    t = jax.nn.gelu(t, approximate=False) * _SCALE
    y = jnp.dot(t, projW_ref[...], preferred_element_type=jnp.float32)
    y = y + projb_ref[...]
    x_ref[...] = y.reshape(_G, _N, _HIDDEN)

    ts = ts_ref[...].reshape(_G * _N)
    onehot = (ts[:, None] == lax.broadcasted_iota(
        jnp.int32, (_G * _N, _MAX_F), 1)).astype(jnp.float32)
    pos = jnp.dot(onehot, posmod_ref[...], preferred_element_type=jnp.float32)
    pos = pos.reshape(_G, _N, _HIDDEN)
    rows = [sess_ref[pl.ds(eid_ref[i * _G + g], 1), :] for g in range(_G)]
    sess = jnp.concatenate(rows, axis=0)
    emb_ref[...] = pos + sess[:, None, :]


def kernel(inputs, inputs_timestamp, inputs_modality, eid, targets, tok_W,
           tok_b, proj_W, proj_b, mod_emb, pos_emb, sess_emb):
    tokWt = tok_W.T
    projWt = proj_W.T
    tokb2 = tok_b.reshape(1, _INPUT_DIM)
    projb2 = proj_b.reshape(1, _HIDDEN)
    posmod = pos_emb + mod_emb[inputs_modality][None, :]
    eid32 = eid.astype(jnp.int32)
    ts32 = inputs_timestamp.astype(jnp.int32)

    grid_spec = pltpu.PrefetchScalarGridSpec(
        num_scalar_prefetch=1,
        grid=(_B // _G,),
        in_specs=[
            pl.BlockSpec((_G, _N, _D), lambda i, *_: (i, 0, 0)),
            pl.BlockSpec((_G, _N), lambda i, *_: (i, 0)),
            pl.BlockSpec((_D, _INPUT_DIM), lambda i, *_: (0, 0)),
            pl.BlockSpec((1, _INPUT_DIM), lambda i, *_: (0, 0)),
            pl.BlockSpec((_INPUT_DIM, _HIDDEN), lambda i, *_: (0, 0)),
            pl.BlockSpec((1, _HIDDEN), lambda i, *_: (0, 0)),
            pl.BlockSpec((_MAX_F, _HIDDEN), lambda i, *_: (0, 0)),
            pl.BlockSpec((_N_SESSIONS, _HIDDEN), lambda i, *_: (0, 0)),
        ],
        out_specs=[
            pl.BlockSpec((_G, _N, _HIDDEN), lambda i, *_: (i, 0, 0)),
            pl.BlockSpec((_G, _N, _HIDDEN), lambda i, *_: (i, 0, 0)),
        ],
    )
    x, emb = pl.pallas_call(
        _body,
        grid_spec=grid_spec,
        out_shape=[jax.ShapeDtypeStruct((_B, _N, _HIDDEN), jnp.float32),
                   jax.ShapeDtypeStruct((_B, _N, _HIDDEN), jnp.float32)],
    )(eid32, inputs, ts32, tokWt, tokb2, projWt, projb2, posmod, sess_emb)
    return (x, emb, targets)


# speedup vs baseline: 5.1268x; 5.1268x over previous
import jax
import jax.numpy as jnp
from jax import lax
from jax.experimental import pallas as pl
from jax.experimental.pallas import tpu as pltpu

_B, _N, _D = 1024, 200, 128
_HIDDEN = 128
_INPUT_DIM = 256
_MAX_F = 200
_N_SESSIONS = 1000
_G = 8
_SCALE = float(_HIDDEN) ** 0.5


def _body(eid_ref, inp_ref, ts_ref, tokW_ref, tokb_ref, projW_ref, projb_ref,
          posmod_ref, sess_ref, x_ref, emb_ref):
    i = pl.program_id(0)
    inp = inp_ref[...].reshape(_G * _N, _D)
    t = jnp.dot(inp, tokW_ref[...], preferred_element_type=jnp.float32)
    t = t + tokb_ref[...]
    t = (0.5 * t * (1.0 + lax.erf(t * (2.0 ** -0.5)))) * _SCALE
    y = jnp.dot(t, projW_ref[...], preferred_element_type=jnp.float32)
    y = y + projb_ref[...]
    x_ref[...] = y.reshape(_G, _N, _HIDDEN)

    ts = ts_ref[...]
    fiota = lax.broadcasted_iota(jnp.int32, (_MAX_F, _N), 0)
    for g in range(_G):
        ohT = (fiota == ts[g, :][None, :]).astype(jnp.float32)
        pos_g = lax.dot_general(ohT, posmod_ref[...],
                                (((0,), (0,)), ((), ())),
                                preferred_element_type=jnp.float32)
        row = sess_ref[pl.ds(eid_ref[i * _G + g], 1), :]
        emb_ref[g, :, :] = pos_g + row


def kernel(inputs, inputs_timestamp, inputs_modality, eid, targets, tok_W,
           tok_b, proj_W, proj_b, mod_emb, pos_emb, sess_emb):
    tokWt = tok_W.T
    projWt = proj_W.T
    tokb2 = tok_b.reshape(1, _INPUT_DIM)
    projb2 = proj_b.reshape(1, _HIDDEN)
    posmod = pos_emb + mod_emb[inputs_modality][None, :]
    eid32 = eid.astype(jnp.int32)
    ts32 = inputs_timestamp.astype(jnp.int32)

    grid_spec = pltpu.PrefetchScalarGridSpec(
        num_scalar_prefetch=1,
        grid=(_B // _G,),
        in_specs=[
            pl.BlockSpec((_G, _N, _D), lambda i, *_: (i, 0, 0)),
            pl.BlockSpec((_G, _N), lambda i, *_: (i, 0)),
            pl.BlockSpec((_D, _INPUT_DIM), lambda i, *_: (0, 0)),
            pl.BlockSpec((1, _INPUT_DIM), lambda i, *_: (0, 0)),
            pl.BlockSpec((_INPUT_DIM, _HIDDEN), lambda i, *_: (0, 0)),
            pl.BlockSpec((1, _HIDDEN), lambda i, *_: (0, 0)),
            pl.BlockSpec((_MAX_F, _HIDDEN), lambda i, *_: (0, 0)),
            pl.BlockSpec((_N_SESSIONS, _HIDDEN), lambda i, *_: (0, 0)),
        ],
        out_specs=[
            pl.BlockSpec((_G, _N, _HIDDEN), lambda i, *_: (i, 0, 0)),
            pl.BlockSpec((_G, _N, _HIDDEN), lambda i, *_: (i, 0, 0)),
        ],
    )
    x, emb = pl.pallas_call(
        _body,
        grid_spec=grid_spec,
        out_shape=[jax.ShapeDtypeStruct((_B, _N, _HIDDEN), jnp.float32),
                   jax.ShapeDtypeStruct((_B, _N, _HIDDEN), jnp.float32)],
    )(eid32, inputs, ts32, tokWt, tokb2, projWt, projb2, posmod, sess_emb)
    return (x, emb, targets)


# speedup vs baseline: 5.1527x; 1.0051x over previous
import jax
import jax.numpy as jnp
from jax import lax
from jax.experimental import pallas as pl
from jax.experimental.pallas import tpu as pltpu

_B, _N, _D = 1024, 200, 128
_HIDDEN = 128
_INPUT_DIM = 256
_MAX_F = 200
_N_SESSIONS = 1000
_G = 8
_SCALE = float(_HIDDEN) ** 0.5


def _body(eid_ref, inp_ref, ts_ref, tokW_ref, tokb_ref, projW_ref, projb_ref,
          posmod_ref, sess_ref, x_ref, emb_ref):
    i = pl.program_id(0)
    inp = inp_ref[...].reshape(_G * _N, _D).astype(jnp.bfloat16)
    t = jnp.dot(inp, tokW_ref[...], preferred_element_type=jnp.float32)
    t = t + tokb_ref[...]
    t = (0.5 * t * (1.0 + lax.erf(t * (2.0 ** -0.5)))) * _SCALE
    y = jnp.dot(t.astype(jnp.bfloat16), projW_ref[...],
                preferred_element_type=jnp.float32)
    y = y + projb_ref[...]
    x_ref[...] = y.reshape(_G, _N, _HIDDEN)

    ts = ts_ref[...]
    fiota = lax.broadcasted_iota(jnp.int32, (_MAX_F, _N), 0)
    for g in range(_G):
        ohT = (fiota == ts[g, :][None, :]).astype(jnp.bfloat16)
        pos_g = lax.dot_general(ohT, posmod_ref[...],
                                (((0,), (0,)), ((), ())),
                                preferred_element_type=jnp.float32)
        row = sess_ref[pl.ds(eid_ref[i * _G + g], 1), :]
        emb_ref[g, :, :] = pos_g + row


def kernel(inputs, inputs_timestamp, inputs_modality, eid, targets, tok_W,
           tok_b, proj_W, proj_b, mod_emb, pos_emb, sess_emb):
    tokWt = tok_W.T.astype(jnp.bfloat16)
    projWt = proj_W.T.astype(jnp.bfloat16)
    tokb2 = tok_b.reshape(1, _INPUT_DIM)
    projb2 = proj_b.reshape(1, _HIDDEN)
    posmod = (pos_emb + mod_emb[inputs_modality][None, :]).astype(jnp.bfloat16)
    eid32 = eid.astype(jnp.int32)
    ts32 = inputs_timestamp.astype(jnp.int32)

    grid_spec = pltpu.PrefetchScalarGridSpec(
        num_scalar_prefetch=1,
        grid=(_B // _G,),
        in_specs=[
            pl.BlockSpec((_G, _N, _D), lambda i, *_: (i, 0, 0)),
            pl.BlockSpec((_G, _N), lambda i, *_: (i, 0)),
            pl.BlockSpec((_D, _INPUT_DIM), lambda i, *_: (0, 0)),
            pl.BlockSpec((1, _INPUT_DIM), lambda i, *_: (0, 0)),
            pl.BlockSpec((_INPUT_DIM, _HIDDEN), lambda i, *_: (0, 0)),
            pl.BlockSpec((1, _HIDDEN), lambda i, *_: (0, 0)),
            pl.BlockSpec((_MAX_F, _HIDDEN), lambda i, *_: (0, 0)),
            pl.BlockSpec((_N_SESSIONS, _HIDDEN), lambda i, *_: (0, 0)),
        ],
        out_specs=[
            pl.BlockSpec((_G, _N, _HIDDEN), lambda i, *_: (i, 0, 0)),
            pl.BlockSpec((_G, _N, _HIDDEN), lambda i, *_: (i, 0, 0)),
        ],
    )
    x, emb = pl.pallas_call(
        _body,
        grid_spec=grid_spec,
        out_shape=[jax.ShapeDtypeStruct((_B, _N, _HIDDEN), jnp.float32),
                   jax.ShapeDtypeStruct((_B, _N, _HIDDEN), jnp.float32)],
    )(eid32, inputs, ts32, tokWt, tokb2, projWt, projb2, posmod, sess_emb)
    return (x, emb, targets)


# speedup vs baseline: 6.1975x; 1.2028x over previous
import jax
import jax.numpy as jnp
from jax import lax
from jax.experimental import pallas as pl
from jax.experimental.pallas import tpu as pltpu

_B, _N, _D = 1024, 200, 128
_HIDDEN = 128
_INPUT_DIM = 256
_MAX_F = 200
_N_SESSIONS = 1000
_G = 16
_SCALE = float(_HIDDEN) ** 0.5


def _body(eid_ref, inp_ref, ts_ref, tokW_ref, tokb_ref, projW_ref, projb_ref,
          posmod_ref, sess_ref, x_ref, emb_ref):
    i = pl.program_id(0)
    inp = inp_ref[...].reshape(_G * _N, _D).astype(jnp.bfloat16)
    t = jnp.dot(inp, tokW_ref[...], preferred_element_type=jnp.float32)
    t = t + tokb_ref[...]
    t = (0.5 * t * (1.0 + lax.erf(t * (2.0 ** -0.5)))) * _SCALE
    y = jnp.dot(t.astype(jnp.bfloat16), projW_ref[...],
                preferred_element_type=jnp.float32)
    y = y + projb_ref[...]
    x_ref[...] = y.reshape(_G, _N, _HIDDEN)

    ts = ts_ref[...]
    fiota = lax.broadcasted_iota(jnp.int32, (_MAX_F, _N), 0)
    for g in range(_G):
        ohT = (fiota == ts[g, :][None, :]).astype(jnp.bfloat16)
        pos_g = lax.dot_general(ohT, posmod_ref[...],
                                (((0,), (0,)), ((), ())),
                                preferred_element_type=jnp.float32)
        row = sess_ref[pl.ds(eid_ref[i * _G + g], 1), :]
        emb_ref[g, :, :] = pos_g + row


def kernel(inputs, inputs_timestamp, inputs_modality, eid, targets, tok_W,
           tok_b, proj_W, proj_b, mod_emb, pos_emb, sess_emb):
    tokWt = tok_W.T.astype(jnp.bfloat16)
    projWt = proj_W.T.astype(jnp.bfloat16)
    tokb2 = tok_b.reshape(1, _INPUT_DIM)
    projb2 = proj_b.reshape(1, _HIDDEN)
    posmod = (pos_emb + mod_emb[inputs_modality][None, :]).astype(jnp.bfloat16)
    eid32 = eid.astype(jnp.int32)
    ts32 = inputs_timestamp.astype(jnp.int32)

    grid_spec = pltpu.PrefetchScalarGridSpec(
        num_scalar_prefetch=1,
        grid=(_B // _G,),
        in_specs=[
            pl.BlockSpec((_G, _N, _D), lambda i, *_: (i, 0, 0)),
            pl.BlockSpec((_G, _N), lambda i, *_: (i, 0)),
            pl.BlockSpec((_D, _INPUT_DIM), lambda i, *_: (0, 0)),
            pl.BlockSpec((1, _INPUT_DIM), lambda i, *_: (0, 0)),
            pl.BlockSpec((_INPUT_DIM, _HIDDEN), lambda i, *_: (0, 0)),
            pl.BlockSpec((1, _HIDDEN), lambda i, *_: (0, 0)),
            pl.BlockSpec((_MAX_F, _HIDDEN), lambda i, *_: (0, 0)),
            pl.BlockSpec((_N_SESSIONS, _HIDDEN), lambda i, *_: (0, 0)),
        ],
        out_specs=[
            pl.BlockSpec((_G, _N, _HIDDEN), lambda i, *_: (i, 0, 0)),
            pl.BlockSpec((_G, _N, _HIDDEN), lambda i, *_: (i, 0, 0)),
        ],
    )
    x, emb = pl.pallas_call(
        _body,
        grid_spec=grid_spec,
        out_shape=[jax.ShapeDtypeStruct((_B, _N, _HIDDEN), jnp.float32),
                   jax.ShapeDtypeStruct((_B, _N, _HIDDEN), jnp.float32)],
    )(eid32, inputs, ts32, tokWt, tokb2, projWt, projb2, posmod, sess_emb)
    return (x, emb, targets)


# speedup vs baseline: 6.8931x; 1.1122x over previous
import jax
import jax.numpy as jnp
from jax import lax
from jax.experimental import pallas as pl
from jax.experimental.pallas import tpu as pltpu

_B, _N, _D = 1024, 200, 128
_HIDDEN = 128
_INPUT_DIM = 256
_MAX_F = 200
_N_SESSIONS = 1000
_G = 32
_SCALE = float(_HIDDEN) ** 0.5


def _body(eid_ref, inp_ref, ts_ref, tokW_ref, tokb_ref, projW_ref, projb_ref,
          posmod_ref, sess_ref, x_ref, emb_ref):
    i = pl.program_id(0)
    inp = inp_ref[...].reshape(_G * _N, _D).astype(jnp.bfloat16)
    t = jnp.dot(inp, tokW_ref[...], preferred_element_type=jnp.float32)
    t = t + tokb_ref[...]
    t = (0.5 * t * (1.0 + lax.erf(t * (2.0 ** -0.5)))) * _SCALE
    y = jnp.dot(t.astype(jnp.bfloat16), projW_ref[...],
                preferred_element_type=jnp.float32)
    y = y + projb_ref[...]
    x_ref[...] = y.reshape(_G, _N, _HIDDEN)

    ts = ts_ref[...]
    fiota = lax.broadcasted_iota(jnp.int32, (_MAX_F, _N), 0)
    for g in range(_G):
        ohT = (fiota == ts[g, :][None, :]).astype(jnp.bfloat16)
        pos_g = lax.dot_general(ohT, posmod_ref[...],
                                (((0,), (0,)), ((), ())),
                                preferred_element_type=jnp.float32)
        row = sess_ref[pl.ds(eid_ref[i * _G + g], 1), :]
        emb_ref[g, :, :] = pos_g + row


def kernel(inputs, inputs_timestamp, inputs_modality, eid, targets, tok_W,
           tok_b, proj_W, proj_b, mod_emb, pos_emb, sess_emb):
    tokWt = tok_W.T.astype(jnp.bfloat16)
    projWt = proj_W.T.astype(jnp.bfloat16)
    tokb2 = tok_b.reshape(1, _INPUT_DIM)
    projb2 = proj_b.reshape(1, _HIDDEN)
    posmod = (pos_emb + mod_emb[inputs_modality][None, :]).astype(jnp.bfloat16)
    eid32 = eid.astype(jnp.int32)
    ts32 = inputs_timestamp.astype(jnp.int32)

    grid_spec = pltpu.PrefetchScalarGridSpec(
        num_scalar_prefetch=1,
        grid=(_B // _G,),
        in_specs=[
            pl.BlockSpec((_G, _N, _D), lambda i, *_: (i, 0, 0)),
            pl.BlockSpec((_G, _N), lambda i, *_: (i, 0)),
            pl.BlockSpec((_D, _INPUT_DIM), lambda i, *_: (0, 0)),
            pl.BlockSpec((1, _INPUT_DIM), lambda i, *_: (0, 0)),
            pl.BlockSpec((_INPUT_DIM, _HIDDEN), lambda i, *_: (0, 0)),
            pl.BlockSpec((1, _HIDDEN), lambda i, *_: (0, 0)),
            pl.BlockSpec((_MAX_F, _HIDDEN), lambda i, *_: (0, 0)),
            pl.BlockSpec((_N_SESSIONS, _HIDDEN), lambda i, *_: (0, 0)),
        ],
        out_specs=[
            pl.BlockSpec((_G, _N, _HIDDEN), lambda i, *_: (i, 0, 0)),
            pl.BlockSpec((_G, _N, _HIDDEN), lambda i, *_: (i, 0, 0)),
        ],
    )
    x, emb = pl.pallas_call(
        _body,
        grid_spec=grid_spec,
        out_shape=[jax.ShapeDtypeStruct((_B, _N, _HIDDEN), jnp.float32),
                   jax.ShapeDtypeStruct((_B, _N, _HIDDEN), jnp.float32)],
    )(eid32, inputs, ts32, tokWt, tokb2, projWt, projb2, posmod, sess_emb)
    return (x, emb, targets)


# speedup vs baseline: 7.3804x; 1.0707x over previous
import jax
import jax.numpy as jnp
from jax import lax
from jax.experimental import pallas as pl
from jax.experimental.pallas import tpu as pltpu

_B, _N, _D = 1024, 200, 128
_HIDDEN = 128
_INPUT_DIM = 256
_MAX_F = 200
_N_SESSIONS = 1000
_G = 64
_SCALE = float(_HIDDEN) ** 0.5


def _body(eid_ref, inp_ref, ts_ref, tokW_ref, tokb_ref, projW_ref, projb_ref,
          posmod_ref, sess_ref, x_ref, emb_ref):
    i = pl.program_id(0)
    inp = inp_ref[...].reshape(_G * _N, _D).astype(jnp.bfloat16)
    t = jnp.dot(inp, tokW_ref[...], preferred_element_type=jnp.float32)
    t = t + tokb_ref[...]
    t = (0.5 * t * (1.0 + lax.erf(t * (2.0 ** -0.5)))) * _SCALE
    y = jnp.dot(t.astype(jnp.bfloat16), projW_ref[...],
                preferred_element_type=jnp.float32)
    y = y + projb_ref[...]
    x_ref[...] = y.reshape(_G, _N, _HIDDEN)

    ts = ts_ref[...]
    fiota = lax.broadcasted_iota(jnp.int32, (_MAX_F, _N), 0)
    for g in range(_G):
        ohT = (fiota == ts[g, :][None, :]).astype(jnp.bfloat16)
        pos_g = lax.dot_general(ohT, posmod_ref[...],
                                (((0,), (0,)), ((), ())),
                                preferred_element_type=jnp.float32)
        row = sess_ref[pl.ds(eid_ref[i * _G + g], 1), :]
        emb_ref[g, :, :] = pos_g + row


def kernel(inputs, inputs_timestamp, inputs_modality, eid, targets, tok_W,
           tok_b, proj_W, proj_b, mod_emb, pos_emb, sess_emb):
    tokWt = tok_W.T.astype(jnp.bfloat16)
    projWt = proj_W.T.astype(jnp.bfloat16)
    tokb2 = tok_b.reshape(1, _INPUT_DIM)
    projb2 = proj_b.reshape(1, _HIDDEN)
    posmod = (pos_emb + mod_emb[inputs_modality][None, :]).astype(jnp.bfloat16)
    eid32 = eid.astype(jnp.int32)
    ts32 = inputs_timestamp.astype(jnp.int32)

    grid_spec = pltpu.PrefetchScalarGridSpec(
        num_scalar_prefetch=1,
        grid=(_B // _G,),
        in_specs=[
            pl.BlockSpec((_G, _N, _D), lambda i, *_: (i, 0, 0)),
            pl.BlockSpec((_G, _N), lambda i, *_: (i, 0)),
            pl.BlockSpec((_D, _INPUT_DIM), lambda i, *_: (0, 0)),
            pl.BlockSpec((1, _INPUT_DIM), lambda i, *_: (0, 0)),
            pl.BlockSpec((_INPUT_DIM, _HIDDEN), lambda i, *_: (0, 0)),
            pl.BlockSpec((1, _HIDDEN), lambda i, *_: (0, 0)),
            pl.BlockSpec((_MAX_F, _HIDDEN), lambda i, *_: (0, 0)),
            pl.BlockSpec((_N_SESSIONS, _HIDDEN), lambda i, *_: (0, 0)),
        ],
        out_specs=[
            pl.BlockSpec((_G, _N, _HIDDEN), lambda i, *_: (i, 0, 0)),
            pl.BlockSpec((_G, _N, _HIDDEN), lambda i, *_: (i, 0, 0)),
        ],
    )
    x, emb = pl.pallas_call(
        _body,
        grid_spec=grid_spec,
        out_shape=[jax.ShapeDtypeStruct((_B, _N, _HIDDEN), jnp.float32),
                   jax.ShapeDtypeStruct((_B, _N, _HIDDEN), jnp.float32)],
    )(eid32, inputs, ts32, tokWt, tokb2, projWt, projb2, posmod, sess_emb)
    return (x, emb, targets)
